# Initial kernel scaffold; baseline (speedup 1.0000x reference)
import jax
import jax.numpy as jnp
from jax.experimental import pallas as pl


def kernel(pos, atom_type, batch, params):
    raise NotImplementedError("write your pallas kernel here")


# speedup vs baseline: 2.2594x; 2.2594x over previous
import functools

import jax
import jax.numpy as jnp
from jax.experimental import pallas as pl

K = 32
G = 16
D = 128
EH = 32
CUTOFF = 10.0
L = 3
BLK = 128
EMBP = 256


def _silu(x):
    return x * jax.nn.sigmoid(x)


def _embed_kernel(at_ref, emb_ref, out_ref):
    at = at_ref[...]
    iota = jax.lax.broadcasted_iota(jnp.int32, (1, EMBP), 1)
    oh = (at == iota).astype(jnp.float32)
    out_ref[...] = jnp.dot(oh, emb_ref[...], preferred_element_type=jnp.float32)


def _topk_kernel(posb_ref, post_ref, bb_ref, br_ref, nbr_ref, elen_ref, *, np_):
    i = pl.program_id(0)
    posb = posb_ref[...]
    post = post_ref[...]
    dx = posb[:, 0:1] - post[0:1, :]
    dy = posb[:, 1:2] - post[1:2, :]
    dz = posb[:, 2:3] - post[2:3, :]
    d2 = dx * dx + dy * dy + dz * dz
    bb = bb_ref[...]
    br = br_ref[...]
    col = jax.lax.broadcasted_iota(jnp.int32, (BLK, np_), 1)
    row = i * BLK + jax.lax.broadcasted_iota(jnp.int32, (BLK, np_), 0)
    valid = (bb == br) & (col != row) & (d2 <= CUTOFF * CUTOFF)
    neg_inf = jnp.float32(-jnp.inf)
    score0 = jnp.where(valid, -d2, neg_inf)

    avail = jnp.full((BLK, np_), True)
    for k in range(K):
        s = jnp.where(avail, score0, neg_inf)
        maxv = jnp.max(s, axis=1, keepdims=True)
        is_max = avail & (s == maxv)
        idx = jnp.min(jnp.where(is_max, col, np_), axis=1, keepdims=True)
        sel = col == idx
        d2sel = jnp.sum(jnp.where(sel, d2, 0.0), axis=1, keepdims=True)
        nbr_ref[:, k:k + 1] = idx
        elen_ref[:, k:k + 1] = jnp.sqrt(d2sel + 1e-12)
        avail = avail & jnp.logical_not(sel)


def _layer_kernel(hb_ref, hf_ref, nbr_ref, elen_ref,
                  ew0_ref, eb0_ref, elnw_ref, elnb_ref, eself_ref,
                  ew1t_ref, eb1_ref, pwt_ref, pb_ref, nw0t_ref, nb0_ref,
                  nln0w_ref, nln0b_ref, nw1t_ref, nb1_ref,
                  nln1w_ref, nln1b_ref, out_ref, *, np_, last):
    hb = hb_ref[...]
    nbr = nbr_ref[...]
    e_all = (elen_ref[...] - 2.7554) / 1.1664
    ew0 = ew0_ref[...]
    eb0 = eb0_ref[...]
    elnw = elnw_ref[...]
    elnb = elnb_ref[...]
    col = jax.lax.broadcasted_iota(jnp.int32, (BLK, np_), 1)

    acc = jnp.zeros((BLK, EH), jnp.float32)
    oh = jnp.zeros((BLK, np_), jnp.float32)
    for k in range(K):
        x = e_all[:, k:k + 1] * ew0 + eb0
        m = jnp.mean(x, axis=1, keepdims=True)
        v = jnp.mean((x - m) ** 2, axis=1, keepdims=True)
        x = (x - m) / jnp.sqrt(v + 1e-5) * elnw + elnb
        acc = acc + _silu(x)
        oh = oh + (col == nbr[:, k:k + 1]).astype(jnp.float32)
    acc = acc + eself_ref[...]
    ea = jnp.dot(acc, ew1t_ref[...], preferred_element_type=jnp.float32)
    ea = ea + (K + 1.0) * eb1_ref[...]
    hsum = jnp.dot(oh, hf_ref[...], preferred_element_type=jnp.float32) + hb
    h0 = jnp.concatenate([hsum, ea], axis=1)
    z = h0 + jnp.dot(hb, pwt_ref[...], preferred_element_type=jnp.float32) + pb_ref[...]
    z = jnp.dot(z, nw0t_ref[...], preferred_element_type=jnp.float32) + nb0_ref[...]
    m = jnp.mean(z, axis=1, keepdims=True)
    v = jnp.mean((z - m) ** 2, axis=1, keepdims=True)
    z = (z - m) / jnp.sqrt(v + 1e-5) * nln0w_ref[...] + nln0b_ref[...]
    z = _silu(z)
    z = jnp.dot(z, nw1t_ref[...], preferred_element_type=jnp.float32) + nb1_ref[...]
    if not last:
        m = jnp.mean(z, axis=1, keepdims=True)
        v = jnp.mean((z - m) ** 2, axis=1, keepdims=True)
        z = (z - m) / jnp.sqrt(v + 1e-5) * nln1w_ref[...] + nln1b_ref[...]
        z = _silu(z)
    out_ref[...] = z


def _gn_kernel(z_ref, bcol_ref, brow_ref, gnw_ref, gnb_ref, gnms_ref, out_ref,
               *, np_):
    z = z_ref[...]
    bcol = bcol_ref[...]
    brow = brow_ref[...]
    gi_r = jax.lax.broadcasted_iota(jnp.int32, (G, np_), 0)
    ohB = (brow == gi_r).astype(jnp.float32)
    gi_c = jax.lax.broadcasted_iota(jnp.int32, (np_, G), 1)
    ohN = (bcol == gi_c).astype(jnp.float32)
    cnt = jnp.maximum(jnp.sum(ohB, axis=1, keepdims=True), 1.0)
    mean = jnp.dot(ohB, z, preferred_element_type=jnp.float32) / cnt
    out = z - gnms_ref[...] * jnp.dot(ohN, mean, preferred_element_type=jnp.float32)
    var = jnp.dot(ohB, out * out, preferred_element_type=jnp.float32) / cnt
    zz = out / jnp.sqrt(jnp.dot(ohN, var, preferred_element_type=jnp.float32) + 1e-5)
    out_ref[...] = _silu(zz * gnw_ref[...] + gnb_ref[...])


def _pool_kernel(h_ref, brow_ref, out_ref, *, np_):
    brow = brow_ref[...]
    gi_r = jax.lax.broadcasted_iota(jnp.int32, (G, np_), 0)
    ohB = (brow == gi_r).astype(jnp.float32)
    cnt = jnp.sum(ohB, axis=1, keepdims=True)
    g = jnp.dot(ohB, h_ref[...], preferred_element_type=jnp.float32) / cnt
    out_ref[...] = jnp.sum(g, axis=1, keepdims=True) / D


def kernel(pos, atom_type, batch, params):
    n = pos.shape[0]
    nb = (n + BLK - 1) // BLK
    np_ = nb * BLK
    padn = np_ - n

    posp = jnp.pad(pos.astype(jnp.float32), ((0, padn), (0, 0)))
    post = posp.T
    bpad = jnp.pad(batch.astype(jnp.int32), (0, padn), constant_values=-1)
    bcol = bpad[:, None]
    brow = bpad[None, :]
    atp = jnp.pad(atom_type.astype(jnp.int32), (0, padn), constant_values=-1)[:, None]
    embp = jnp.pad(params["emb"].astype(jnp.float32),
                   ((0, EMBP - params["emb"].shape[0]), (0, 0)))

    h = pl.pallas_call(
        _embed_kernel,
        grid=(nb,),
        in_specs=[pl.BlockSpec((BLK, 1), lambda i: (i, 0)),
                  pl.BlockSpec((EMBP, D), lambda i: (0, 0))],
        out_specs=pl.BlockSpec((BLK, D), lambda i: (i, 0)),
        out_shape=jax.ShapeDtypeStruct((np_, D), jnp.float32),
    )(atp, embp)

    nbr, elen = pl.pallas_call(
        functools.partial(_topk_kernel, np_=np_),
        grid=(nb,),
        in_specs=[pl.BlockSpec((BLK, 3), lambda i: (i, 0)),
                  pl.BlockSpec((3, np_), lambda i: (0, 0)),
                  pl.BlockSpec((BLK, 1), lambda i: (i, 0)),
                  pl.BlockSpec((1, np_), lambda i: (0, 0))],
        out_specs=[pl.BlockSpec((BLK, K), lambda i: (i, 0)),
                   pl.BlockSpec((BLK, K), lambda i: (i, 0))],
        out_shape=[jax.ShapeDtypeStruct((np_, K), jnp.int32),
                   jax.ShapeDtypeStruct((np_, K), jnp.float32)],
    )(posp, post, bcol, brow)

    e_self = (jnp.float32(1e-6) - 2.7554) / 1.1664
    for l in range(L):
        p = params["layer%d" % l]
        x = e_self * p["ew0"][:, 0] + p["eb0"]
        m = x.mean()
        v = ((x - m) ** 2).mean()
        x = (x - m) / jnp.sqrt(v + 1e-5) * p["elnw"] + p["elnb"]
        eself = _silu(x)[None, :]
        last = (l + 1 == L)
        full = lambda shape: pl.BlockSpec(shape, lambda i: (0, 0))
        h = pl.pallas_call(
            functools.partial(_layer_kernel, np_=np_, last=last),
            grid=(nb,),
            in_specs=[pl.BlockSpec((BLK, D), lambda i: (i, 0)),
                      full((np_, D)),
                      pl.BlockSpec((BLK, K), lambda i: (i, 0)),
                      pl.BlockSpec((BLK, K), lambda i: (i, 0)),
                      full((1, EH)), full((1, EH)), full((1, EH)),
                      full((1, EH)), full((1, EH)),
                      full((EH, EH)), full((1, EH)),
                      full((D, D + EH)), full((1, D + EH)),
                      full((D + EH, D)), full((1, D)),
                      full((1, D)), full((1, D)),
                      full((D, D)), full((1, D)),
                      full((1, D)), full((1, D))],
            out_specs=pl.BlockSpec((BLK, D), lambda i: (i, 0)),
            out_shape=jax.ShapeDtypeStruct((np_, D), jnp.float32),
        )(h, h, nbr, elen,
          p["ew0"][:, 0][None, :], p["eb0"][None, :],
          p["elnw"][None, :], p["elnb"][None, :], eself,
          p["ew1"].T, p["eb1"][None, :],
          p["pw"].T, p["pb"][None, :],
          p["nw0"].T, p["nb0"][None, :],
          p["nln0w"][None, :], p["nln0b"][None, :],
          p["nw1"].T, p["nb1"][None, :],
          p["nln1w"][None, :], p["nln1b"][None, :])
        if not last:
            h = pl.pallas_call(
                functools.partial(_gn_kernel, np_=np_),
                out_shape=jax.ShapeDtypeStruct((np_, D), jnp.float32),
            )(h, bcol, brow, p["gnw"][None, :], p["gnb"][None, :],
              p["gnms"][None, :])

    g = pl.pallas_call(
        functools.partial(_pool_kernel, np_=np_),
        out_shape=jax.ShapeDtypeStruct((G, 1), jnp.float32),
    )(h, brow)
    return g[:, 0]


# speedup vs baseline: 3.2843x; 1.4537x over previous
import functools

import jax
import jax.numpy as jnp
from jax.experimental import pallas as pl

K = 32
G = 16
D = 128
EH = 32
CUTOFF = 10.0
L = 3
BLK = 128
EMBP = 256


def _silu(x):
    return x * jax.nn.sigmoid(x)


def _embed_kernel(at_ref, emb_ref, out_ref):
    at = at_ref[...]
    iota = jax.lax.broadcasted_iota(jnp.int32, (1, EMBP), 1)
    oh = (at == iota).astype(jnp.float32)
    out_ref[...] = jnp.dot(oh, emb_ref[...], preferred_element_type=jnp.float32)


def _topk_kernel(posb_ref, post_ref, bb_ref, br_ref, adj_ref, elen_ref, *, np_):
    i = pl.program_id(0)
    posb = posb_ref[...]
    post = post_ref[...]
    dotp = jnp.dot(posb.astype(jnp.bfloat16), post.astype(jnp.bfloat16),
                   preferred_element_type=jnp.float32)
    sqb = jnp.sum(posb * posb, axis=1, keepdims=True)
    sqr = jnp.sum(post * post, axis=0, keepdims=True)
    d2m = sqb + sqr - 2.0 * dotp
    dx = posb[:, 0:1] - post[0:1, :]
    dy = posb[:, 1:2] - post[1:2, :]
    dz = posb[:, 2:3] - post[2:3, :]
    d2 = dx * dx + dy * dy + dz * dz
    bb = bb_ref[...]
    br = br_ref[...]
    col = jax.lax.broadcasted_iota(jnp.int32, (BLK, np_), 1)
    row = i * BLK + jax.lax.broadcasted_iota(jnp.int32, (BLK, np_), 0)
    valid = (bb == br) & (col != row) & (d2m <= CUTOFF * CUTOFF)
    score = jnp.where(valid, -d2m, -1e6 - col.astype(jnp.float32))
    neg_inf = jnp.float32(-jnp.inf)
    oh = jnp.zeros((BLK, np_), jnp.float32)
    for k in range(K):
        maxv = jnp.max(score, axis=1, keepdims=True)
        idx = jnp.min(jnp.where(score == maxv, col, np_), axis=1, keepdims=True)
        sel = col == idx
        d2sel = jax.lax.cond(
            jnp.any(maxv < -1e5),
            lambda: jnp.where(maxv < -1e5,
                              jnp.sum(jnp.where(sel, d2, 0.0), axis=1,
                                      keepdims=True),
                              -maxv),
            lambda: -maxv)
        elen_ref[:, k:k + 1] = jnp.sqrt(jnp.maximum(d2sel, 0.0) + 1e-12)
        oh = oh + sel.astype(jnp.float32)
        score = jnp.where(sel, neg_inf, score)
    adj_ref[...] = oh


def _layer_kernel(hb_ref, hf_ref, adj_ref, elen_ref,
                  ew0_ref, eb0_ref, elnw_ref, elnb_ref, eself_ref,
                  ew1t_ref, eb1_ref, pwt_ref, pb_ref, nw0t_ref, nb0_ref,
                  nln0w_ref, nln0b_ref, nw1t_ref, nb1_ref,
                  nln1w_ref, nln1b_ref, out_ref, *, np_, last):
    hb = hb_ref[...]
    oh = adj_ref[...]
    e_all = (elen_ref[...] - 2.7554) / 1.1664
    ew0 = ew0_ref[...]
    eb0 = eb0_ref[...]
    elnw = elnw_ref[...]
    elnb = elnb_ref[...]

    acc = jnp.zeros((BLK, EH), jnp.float32)
    for k in range(K):
        x = e_all[:, k:k + 1] * ew0 + eb0
        m = jnp.mean(x, axis=1, keepdims=True)
        v = jnp.mean((x - m) ** 2, axis=1, keepdims=True)
        x = (x - m) / jnp.sqrt(v + 1e-5) * elnw + elnb
        acc = acc + _silu(x)
    acc = acc + eself_ref[...]
    ea = jnp.dot(acc, ew1t_ref[...], preferred_element_type=jnp.float32)
    ea = ea + (K + 1.0) * eb1_ref[...]
    hsum = jnp.dot(oh, hf_ref[...], preferred_element_type=jnp.float32) + hb
    h0 = jnp.concatenate([hsum, ea], axis=1)
    z = h0 + jnp.dot(hb, pwt_ref[...], preferred_element_type=jnp.float32) + pb_ref[...]
    z = jnp.dot(z, nw0t_ref[...], preferred_element_type=jnp.float32) + nb0_ref[...]
    m = jnp.mean(z, axis=1, keepdims=True)
    v = jnp.mean((z - m) ** 2, axis=1, keepdims=True)
    z = (z - m) / jnp.sqrt(v + 1e-5) * nln0w_ref[...] + nln0b_ref[...]
    z = _silu(z)
    z = jnp.dot(z, nw1t_ref[...], preferred_element_type=jnp.float32) + nb1_ref[...]
    if not last:
        m = jnp.mean(z, axis=1, keepdims=True)
        v = jnp.mean((z - m) ** 2, axis=1, keepdims=True)
        z = (z - m) / jnp.sqrt(v + 1e-5) * nln1w_ref[...] + nln1b_ref[...]
        z = _silu(z)
    out_ref[...] = z


def _gn_kernel(z_ref, bcol_ref, brow_ref, gnw_ref, gnb_ref, gnms_ref, out_ref,
               *, np_):
    z = z_ref[...]
    bcol = bcol_ref[...]
    brow = brow_ref[...]
    gi_r = jax.lax.broadcasted_iota(jnp.int32, (G, np_), 0)
    ohB = (brow == gi_r).astype(jnp.float32)
    gi_c = jax.lax.broadcasted_iota(jnp.int32, (np_, G), 1)
    ohN = (bcol == gi_c).astype(jnp.float32)
    cnt = jnp.maximum(jnp.sum(ohB, axis=1, keepdims=True), 1.0)
    mean = jnp.dot(ohB, z, preferred_element_type=jnp.float32) / cnt
    out = z - gnms_ref[...] * jnp.dot(ohN, mean, preferred_element_type=jnp.float32)
    var = jnp.dot(ohB, out * out, preferred_element_type=jnp.float32) / cnt
    zz = out / jnp.sqrt(jnp.dot(ohN, var, preferred_element_type=jnp.float32) + 1e-5)
    out_ref[...] = _silu(zz * gnw_ref[...] + gnb_ref[...])


def _pool_kernel(h_ref, brow_ref, out_ref, *, np_):
    brow = brow_ref[...]
    gi_r = jax.lax.broadcasted_iota(jnp.int32, (G, np_), 0)
    ohB = (brow == gi_r).astype(jnp.float32)
    cnt = jnp.sum(ohB, axis=1, keepdims=True)
    g = jnp.dot(ohB, h_ref[...], preferred_element_type=jnp.float32) / cnt
    out_ref[...] = jnp.sum(g, axis=1, keepdims=True) / D


def kernel(pos, atom_type, batch, params):
    n = pos.shape[0]
    nb = (n + BLK - 1) // BLK
    np_ = nb * BLK
    padn = np_ - n

    posp = jnp.pad(pos.astype(jnp.float32), ((0, padn), (0, 0)))
    post = posp.T
    bpad = jnp.pad(batch.astype(jnp.int32), (0, padn), constant_values=-1)
    bcol = bpad[:, None]
    brow = bpad[None, :]
    atp = jnp.pad(atom_type.astype(jnp.int32), (0, padn), constant_values=-1)[:, None]
    embp = jnp.pad(params["emb"].astype(jnp.float32),
                   ((0, EMBP - params["emb"].shape[0]), (0, 0)))

    h = pl.pallas_call(
        _embed_kernel,
        grid=(nb,),
        in_specs=[pl.BlockSpec((BLK, 1), lambda i: (i, 0)),
                  pl.BlockSpec((EMBP, D), lambda i: (0, 0))],
        out_specs=pl.BlockSpec((BLK, D), lambda i: (i, 0)),
        out_shape=jax.ShapeDtypeStruct((np_, D), jnp.float32),
    )(atp, embp)

    adj, elen = pl.pallas_call(
        functools.partial(_topk_kernel, np_=np_),
        grid=(nb,),
        in_specs=[pl.BlockSpec((BLK, 3), lambda i: (i, 0)),
                  pl.BlockSpec((3, np_), lambda i: (0, 0)),
                  pl.BlockSpec((BLK, 1), lambda i: (i, 0)),
                  pl.BlockSpec((1, np_), lambda i: (0, 0))],
        out_specs=[pl.BlockSpec((BLK, np_), lambda i: (i, 0)),
                   pl.BlockSpec((BLK, K), lambda i: (i, 0))],
        out_shape=[jax.ShapeDtypeStruct((np_, np_), jnp.float32),
                   jax.ShapeDtypeStruct((np_, K), jnp.float32)],
    )(posp, post, bcol, brow)

    e_self = (jnp.float32(1e-6) - 2.7554) / 1.1664
    for l in range(L):
        p = params["layer%d" % l]
        x = e_self * p["ew0"][:, 0] + p["eb0"]
        m = x.mean()
        v = ((x - m) ** 2).mean()
        x = (x - m) / jnp.sqrt(v + 1e-5) * p["elnw"] + p["elnb"]
        eself = _silu(x)[None, :]
        last = (l + 1 == L)
        full = lambda shape: pl.BlockSpec(shape, lambda i: (0, 0))
        h = pl.pallas_call(
            functools.partial(_layer_kernel, np_=np_, last=last),
            grid=(nb,),
            in_specs=[pl.BlockSpec((BLK, D), lambda i: (i, 0)),
                      full((np_, D)),
                      pl.BlockSpec((BLK, np_), lambda i: (i, 0)),
                      pl.BlockSpec((BLK, K), lambda i: (i, 0)),
                      full((1, EH)), full((1, EH)), full((1, EH)),
                      full((1, EH)), full((1, EH)),
                      full((EH, EH)), full((1, EH)),
                      full((D, D + EH)), full((1, D + EH)),
                      full((D + EH, D)), full((1, D)),
                      full((1, D)), full((1, D)),
                      full((D, D)), full((1, D)),
                      full((1, D)), full((1, D))],
            out_specs=pl.BlockSpec((BLK, D), lambda i: (i, 0)),
            out_shape=jax.ShapeDtypeStruct((np_, D), jnp.float32),
        )(h, h, adj, elen,
          p["ew0"][:, 0][None, :], p["eb0"][None, :],
          p["elnw"][None, :], p["elnb"][None, :], eself,
          p["ew1"].T, p["eb1"][None, :],
          p["pw"].T, p["pb"][None, :],
          p["nw0"].T, p["nb0"][None, :],
          p["nln0w"][None, :], p["nln0b"][None, :],
          p["nw1"].T, p["nb1"][None, :],
          p["nln1w"][None, :], p["nln1b"][None, :])
        if not last:
            h = pl.pallas_call(
                functools.partial(_gn_kernel, np_=np_),
                out_shape=jax.ShapeDtypeStruct((np_, D), jnp.float32),
            )(h, bcol, brow, p["gnw"][None, :], p["gnb"][None, :],
              p["gnms"][None, :])

    g = pl.pallas_call(
        functools.partial(_pool_kernel, np_=np_),
        out_shape=jax.ShapeDtypeStruct((G, 1), jnp.float32),
    )(h, brow)
    return g[:, 0]
